# baseline (speedup 1.0000x reference)
import functools
import math

import jax
import jax.numpy as jnp
from jax import lax
from jax.experimental import pallas as pl
from jax.experimental.pallas import tpu as pltpu
from jax.experimental.pallas import tpu_sc as plsc

EMBED = 64
SCALE = 8.0
NC = 2
NS = 16
NW = NC * NS
CHUNK = 128
LANES = 16
NBUF = 4


def _make_gather(total, n_chunks):
    mesh = plsc.VectorSubcoreMesh(core_axis_name="c", subcore_axis_name="s")
    n_groups = n_chunks // NBUF

    @functools.partial(
        pl.kernel,
        mesh=mesh,
        compiler_params=pltpu.CompilerParams(use_tc_tiling_on_sc=False),
        out_type=jax.ShapeDtypeStruct((total, EMBED), jnp.float32),
        scratch_types=(
            [pltpu.VMEM((n_chunks, CHUNK), jnp.int32)]
            + [pltpu.VMEM((CHUNK, EMBED), jnp.float32) for _ in range(2 * NBUF)]
            + [pltpu.SemaphoreType.DMA for _ in range(2 * NBUF)]
        ),
    )
    def gather_kernel(idx_hbm, table_hbm, out_hbm, idx_v, *bufs_and_sems):
        a_bufs = bufs_and_sems[:NBUF]
        b_bufs = bufs_and_sems[NBUF:2 * NBUF]
        g_sems = bufs_and_sems[2 * NBUF:3 * NBUF]
        o_sems = bufs_and_sems[3 * NBUF:]
        wid = lax.axis_index("s") * NC + lax.axis_index("c")
        base = wid * (n_chunks * CHUNK)
        pltpu.sync_copy(idx_hbm.at[wid], idx_v)

        for b in range(NBUF):
            pltpu.async_copy(table_hbm.at[idx_v.at[b]], a_bufs[b], g_sems[b])

        def group_body(g, carry):
            for b in range(NBUF):
                j = g * NBUF + b
                pltpu.make_async_copy(
                    table_hbm.at[idx_v.at[j]], a_bufs[b], g_sems[b]
                ).wait()
                @pl.when(g > 0)
                def _():
                    pltpu.make_async_copy(
                        b_bufs[b], out_hbm.at[pl.ds(base + j * CHUNK, CHUNK)],
                        o_sems[b],
                    ).wait()

                def scale_body(r, c2):
                    for cc in range(EMBED // LANES):
                        sl = pl.ds(cc * LANES, LANES)
                        b_bufs[b][r, sl] = a_bufs[b][r, sl] * SCALE
                    return c2

                lax.fori_loop(0, CHUNK, scale_body, 0, unroll=4)
                pltpu.async_copy(
                    b_bufs[b], out_hbm.at[pl.ds(base + j * CHUNK, CHUNK)],
                    o_sems[b],
                )

                @pl.when(j + NBUF < n_chunks)
                def _():
                    pltpu.async_copy(
                        table_hbm.at[idx_v.at[j + NBUF]], a_bufs[b], g_sems[b]
                    )

            return carry

        lax.fori_loop(0, n_groups, group_body, 0)

        for b in range(NBUF):
            j = (n_groups - 1) * NBUF + b
            pltpu.make_async_copy(
                b_bufs[b], out_hbm.at[pl.ds(base + j * CHUNK, CHUNK)], o_sems[b]
            ).wait()

    return gather_kernel


def kernel(input_token, table):
    batch, hist = input_token.shape
    total = batch * hist
    n_chunks = total // (NW * CHUNK)
    idx = input_token.reshape(NW, n_chunks, CHUNK).astype(jnp.int32)
    out = _make_gather(total, n_chunks)(idx, table)
    return out.reshape(batch, hist, EMBED)


# speedup vs baseline: 1.1159x; 1.1159x over previous
import functools
import math

import jax
import jax.numpy as jnp
from jax import lax
from jax.experimental import pallas as pl
from jax.experimental.pallas import tpu as pltpu
from jax.experimental.pallas import tpu_sc as plsc

EMBED = 64
SCALE = 8.0
NC = 2
NS = 16
NW = NC * NS
LANES = 16


def _make_gather(batch, hist):
    mesh = plsc.VectorSubcoreMesh(core_axis_name="c", subcore_axis_name="s")
    rows_per_w = batch // NW

    @functools.partial(
        pl.kernel,
        mesh=mesh,
        compiler_params=pltpu.CompilerParams(use_tc_tiling_on_sc=False),
        out_type=jax.ShapeDtypeStruct((batch, hist, EMBED), jnp.float32),
        scratch_types=[
            pltpu.VMEM((rows_per_w, hist), jnp.int32),
            pltpu.VMEM((hist, EMBED), jnp.float32),
            pltpu.VMEM((hist, EMBED), jnp.float32),
            pltpu.SemaphoreType.DMA,
            pltpu.SemaphoreType.DMA,
        ],
    )
    def gather_kernel(idx_hbm, table_hbm, out_hbm, idx_v, buf0, buf1, sem0, sem1):
        wid = lax.axis_index("s") * NC + lax.axis_index("c")
        row0 = wid * rows_per_w
        pltpu.sync_copy(idx_hbm.at[pl.ds(row0, rows_per_w)], idx_v)

        bufs = (buf0, buf1)
        sems = (sem0, sem1)
        pltpu.async_copy(table_hbm.at[idx_v.at[0]], buf0, sem0)

        def process(r, buf, sem, nbuf, nsem):
            @pl.when(r + 1 < rows_per_w)
            def _():
                pltpu.async_copy(table_hbm.at[idx_v.at[r + 1]], nbuf, nsem)

            pltpu.make_async_copy(table_hbm.at[idx_v.at[r]], buf, sem).wait()

            def scale_body(h, c2):
                for cc in range(EMBED // LANES):
                    sl = pl.ds(cc * LANES, LANES)
                    buf[h, sl] = buf[h, sl] * SCALE
                return c2

            lax.fori_loop(0, hist, scale_body, 0, unroll=2)
            pltpu.sync_copy(buf, out_hbm.at[row0 + r])

        def pair_body(g, carry):
            process(2 * g, buf0, sem0, buf1, sem1)
            process(2 * g + 1, buf1, sem1, buf0, sem0)
            return carry

        lax.fori_loop(0, rows_per_w // 2, pair_body, 0)

    return gather_kernel


def kernel(input_token, table):
    batch, hist = input_token.shape
    if input_token.dtype != jnp.int32:
        input_token = input_token.astype(jnp.int32)
    return _make_gather(batch, hist)(input_token, table)


# speedup vs baseline: 1.1694x; 1.0480x over previous
import functools
import math

import jax
import jax.numpy as jnp
from jax import lax
from jax.experimental import pallas as pl
from jax.experimental.pallas import tpu as pltpu
from jax.experimental.pallas import tpu_sc as plsc

EMBED = 64
SCALE = 8.0
NC = 2
NS = 16
NW = NC * NS
LANES = 16


def _make_gather(batch, hist):
    mesh = plsc.VectorSubcoreMesh(core_axis_name="c", subcore_axis_name="s")
    rows_per_w = batch // NW

    @functools.partial(
        pl.kernel,
        mesh=mesh,
        compiler_params=pltpu.CompilerParams(use_tc_tiling_on_sc=False),
        out_type=jax.ShapeDtypeStruct((batch, hist, EMBED), jnp.float32),
        scratch_types=[
            pltpu.VMEM((rows_per_w, hist), jnp.int32),
            pltpu.VMEM((hist, EMBED), jnp.float32),
            pltpu.VMEM((hist, EMBED), jnp.float32),
            pltpu.SemaphoreType.DMA,
            pltpu.SemaphoreType.DMA,
        ],
    )
    def gather_kernel(idx_hbm, table_hbm, out_hbm, idx_v, buf0, buf1, sem0, sem1):
        wid = lax.axis_index("s") * NC + lax.axis_index("c")
        row0 = wid * rows_per_w
        pltpu.sync_copy(idx_hbm.at[pl.ds(row0, rows_per_w)], idx_v)

        bufs = (buf0, buf1)
        sems = (sem0, sem1)
        pltpu.async_copy(table_hbm.at[idx_v.at[0]], buf0, sem0)

        def process(r, buf, sem, nbuf, nsem):
            @pl.when(r + 1 < rows_per_w)
            def _():
                pltpu.async_copy(table_hbm.at[idx_v.at[r + 1]], nbuf, nsem)

            pltpu.make_async_copy(table_hbm.at[idx_v.at[r]], buf, sem).wait()

            def scale_body(h, c2):
                for cc in range(EMBED // LANES):
                    sl = pl.ds(cc * LANES, LANES)
                    buf[h, sl] = buf[h, sl] * SCALE
                return c2

            lax.fori_loop(0, hist, scale_body, 0, unroll=2)
            pltpu.sync_copy(buf, out_hbm.at[row0 + r])

        def pair_body(g, carry):
            process(2 * g, buf0, sem0, buf1, sem1)
            process(2 * g + 1, buf1, sem1, buf0, sem0)
            return carry

        lax.fori_loop(0, rows_per_w // 2, pair_body, 0)

    return gather_kernel


def kernel(input_token, table):
    batch, hist = input_token.shape
    vocab = table.shape[0]
    table_p = jnp.pad(table, ((0, 0), (0, 128 - EMBED))).reshape(2 * vocab, EMBED)
    idx2 = input_token.astype(jnp.int32) * 2
    return _make_gather(batch, hist)(idx2, table_p)


# speedup vs baseline: 1.9574x; 1.6738x over previous
import functools
import math

import jax
import jax.numpy as jnp
from jax import lax
from jax.experimental import pallas as pl
from jax.experimental.pallas import tpu as pltpu
from jax.experimental.pallas import tpu_sc as plsc

EMBED = 64
SCALE = 8.0
NC = 2
NS = 16
NW = NC * NS
LANES = 16
TB = 128


def _make_gather(batch, hist):
    mesh = plsc.VectorSubcoreMesh(core_axis_name="c", subcore_axis_name="s")
    nbt = batch // TB
    bt_per_w = nbt // NW
    rows_w = bt_per_w * TB
    units = bt_per_w * hist

    @functools.partial(
        pl.kernel,
        mesh=mesh,
        compiler_params=pltpu.CompilerParams(use_tc_tiling_on_sc=False),
        out_type=jax.ShapeDtypeStruct((hist, 8, nbt, 8, TB), jnp.float32),
        scratch_types=[
            pltpu.VMEM((hist, rows_w), jnp.int32),
            pltpu.VMEM((TB, EMBED), jnp.float32),
            pltpu.VMEM((TB, EMBED), jnp.float32),
            pltpu.VMEM((8, 8, TB), jnp.float32),
            pltpu.VMEM((8, 8, TB), jnp.float32),
            pltpu.SemaphoreType.DMA,
            pltpu.SemaphoreType.DMA,
            pltpu.SemaphoreType.DMA,
            pltpu.SemaphoreType.DMA,
        ],
    )
    def gather_kernel(idx_hbm, table_hbm, out_hbm, idx_v,
                      gb0, gb1, tb0, tb1, gs0, gs1, os0, os1):
        wid = lax.axis_index("s") * NC + lax.axis_index("c")
        bt0 = wid * bt_per_w
        pltpu.sync_copy(idx_hbm.at[:, pl.ds(bt0 * TB, rows_w)], idx_v)

        def fire(u, gb, gs):
            bt = u // hist
            h = u % hist
            pltpu.async_copy(
                table_hbm.at[idx_v.at[h, pl.ds(bt * TB, TB)]], gb, gs
            )

        def process(u, gb, gs, tb, osem, ogb, ogs):
            fire(jnp.minimum(u + 1, units - 1), ogb, ogs)

            pltpu.make_async_copy(
                table_hbm.at[idx_v.at[0, pl.ds(0, TB)]], gb, gs
            ).wait()

            bt = u // hist
            h = u % hist

            @pl.when(u >= 2)
            def _():
                pltpu.make_async_copy(
                    tb, out_hbm.at[h, :, bt0 + bt], osem
                ).wait()

            def tc_body(tc, carry):
                t0 = tc * LANES
                iota = lax.iota(jnp.int32, LANES)
                for ec in range(EMBED // LANES):
                    e0 = ec * LANES
                    v = [gb[t0 + i, pl.ds(e0, LANES)] * SCALE
                         for i in range(LANES)]
                    for d in (1, 2, 4, 8):
                        mask = (iota & d) == 0
                        rm = (iota + (LANES - d)) % LANES
                        rp = (iota + d) % LANES
                        for i in range(LANES):
                            if i & d:
                                continue
                            a, b = v[i], v[i + d]
                            br = b.at[rm].get(mode="promise_in_bounds")
                            ar = a.at[rp].get(mode="promise_in_bounds")
                            v[i] = jnp.where(mask, a, br)
                            v[i + d] = jnp.where(mask, ar, b)
                    for j in range(LANES):
                        e = e0 + j
                        tb[e // 8, e % 8, pl.ds(t0, LANES)] = v[j]
                return carry

            lax.fori_loop(0, TB // LANES, tc_body, 0)
            pltpu.async_copy(tb, out_hbm.at[h, :, bt0 + bt], osem)

        fire(0, gb0, gs0)

        def pair_body(g, carry):
            process(2 * g, gb0, gs0, tb0, os0, gb1, gs1)
            process(2 * g + 1, gb1, gs1, tb1, os1, gb0, gs0)
            return carry

        lax.fori_loop(0, units // 2, pair_body, 0)

        pltpu.make_async_copy(
            table_hbm.at[idx_v.at[0, pl.ds(0, TB)]], gb0, gs0
        ).wait()
        for tb, osem, u in ((tb0, os0, units - 2), (tb1, os1, units - 1)):
            bt = u // hist
            h = u % hist
            pltpu.make_async_copy(tb, out_hbm.at[h, :, bt0 + bt], osem).wait()

    return gather_kernel


def kernel(input_token, table):
    batch, hist = input_token.shape
    vocab = table.shape[0]
    nbt = batch // TB
    table_p = jnp.pad(table, ((0, 0), (0, 128 - EMBED))).reshape(2 * vocab, EMBED)
    idx2t = (input_token.astype(jnp.int32) * 2).T
    out5d = _make_gather(batch, hist)(idx2t, table_p)
    return out5d.transpose(2, 4, 0, 1, 3).reshape(batch, hist, EMBED)
